# baseline (speedup 1.0000x reference)
import jax
import jax.numpy as jnp
from jax import lax
from jax.experimental import pallas as pl
from jax.experimental.pallas import tpu as pltpu
from jax.experimental.pallas import tpu_sc as plsc


def _lookup_body(idx_hbm, table_hbm, out_hbm, idx_v, row_v, sem):
    c = lax.axis_index("c")
    s = lax.axis_index("s")

    @pl.when(jnp.logical_and(c == 0, s == 0))
    def _():
        pltpu.sync_copy(idx_hbm, idx_v)
        pltpu.async_copy(table_hbm.at[idx_v], row_v, sem).wait()
        pltpu.sync_copy(row_v, out_hbm)


def kernel(token, table):
    emb = table.shape[1]
    idx = jnp.asarray(token, jnp.int32).reshape((1,))
    mesh = plsc.VectorSubcoreMesh(core_axis_name="c", subcore_axis_name="s")
    k = pl.kernel(
        _lookup_body,
        out_type=jax.ShapeDtypeStruct((1, emb), jnp.float32),
        mesh=mesh,
        scratch_types=[
            pltpu.VMEM((1,), jnp.int32),
            pltpu.VMEM((1, emb), jnp.float32),
            pltpu.SemaphoreType.DMA,
        ],
    )
    out = k(idx, table)
    return jnp.squeeze(out, axis=0)


# speedup vs baseline: 1.1050x; 1.1050x over previous
import jax
import jax.numpy as jnp
from jax import lax
from jax.experimental import pallas as pl
from jax.experimental.pallas import tpu as pltpu
from jax.experimental.pallas import tpu_sc as plsc


def _lookup_body(idx_hbm, table_hbm, out_hbm, idx_v, row_v, sem):
    c = lax.axis_index("c")
    s = lax.axis_index("s")

    @pl.when(jnp.logical_and(c == 0, s == 0))
    def _():
        pltpu.sync_copy(idx_hbm, idx_v)
        pltpu.async_copy(table_hbm.at[idx_v], row_v, sem).wait()
        pltpu.sync_copy(row_v, out_hbm)


def kernel(token, table):
    emb = table.shape[1]
    idx = jnp.asarray(token, jnp.int32).reshape((1,))
    mesh = plsc.VectorSubcoreMesh(
        core_axis_name="c", subcore_axis_name="s", num_cores=1, num_subcores=1
    )
    k = pl.kernel(
        _lookup_body,
        out_type=jax.ShapeDtypeStruct((1, emb), jnp.float32),
        mesh=mesh,
        scratch_types=[
            pltpu.VMEM((1,), jnp.int32),
            pltpu.VMEM((1, emb), jnp.float32),
            pltpu.SemaphoreType.DMA,
        ],
    )
    out = k(idx, table)
    return jnp.squeeze(out, axis=0)


# speedup vs baseline: 10.9662x; 9.9241x over previous
import jax
import jax.numpy as jnp
from jax.experimental import pallas as pl
from jax.experimental.pallas import tpu as pltpu


def _lookup_body(tok_ref, tile_ref, out_ref):
    r = tok_ref[0] % 8
    out_ref[...] = tile_ref[pl.ds(r, 1), :]


def kernel(token, table):
    emb = table.shape[1]
    tok = jnp.asarray(token, jnp.int32).reshape((1,))
    grid_spec = pltpu.PrefetchScalarGridSpec(
        num_scalar_prefetch=1,
        grid=(1,),
        in_specs=[pl.BlockSpec((8, emb), lambda i, tok: (tok[0] // 8, 0))],
        out_specs=pl.BlockSpec((1, emb), lambda i, tok: (0, 0)),
    )
    out = pl.pallas_call(
        _lookup_body,
        grid_spec=grid_spec,
        out_shape=jax.ShapeDtypeStruct((1, emb), jnp.float32),
    )(tok, table)
    return jnp.squeeze(out, axis=0)
